# Initial kernel scaffold; baseline (speedup 1.0000x reference)
import functools

import jax
import jax.numpy as jnp
from jax import lax
from jax.experimental import pallas as pl
from jax.experimental.pallas import tpu as pltpu
from jax.experimental.pallas import tpu_sc as plsc

VOCAB = 30522
D = 64
B = 16384
L = 200

NC = 2
NS = 16
LANES = 16
NW = NC * NS
ROWS_W = B // NW
TOK_W = ROWS_W * L
CHUNK = 128
IDXBLK = 50
NBLK = TOK_W // (IDXBLK * CHUNK)
ROWS_CORE = B // NC
INV_L = 1.0 / L


def _sc_embed_mean(ids2d, table):
    mesh = plsc.VectorSubcoreMesh(core_axis_name="c", subcore_axis_name="s")

    @functools.partial(
        pl.kernel,
        mesh=mesh,
        out_type=jax.ShapeDtypeStruct((B, D), jnp.float32),
        scratch_types=[
            pltpu.VMEM((IDXBLK, CHUNK), jnp.int32),
            pltpu.VMEM((CHUNK, D), jnp.float32),
            pltpu.VMEM((CHUNK,), jnp.int32),
            pltpu.VMEM_SHARED((ROWS_CORE, D), jnp.float32),
        ],
    )
    def k(ids_hbm, table_hbm, out_hbm, tidx_v, rows_v, didx_v, acc_sh):
        c = lax.axis_index("c")
        s = lax.axis_index("s")
        row0_l = s * ROWS_W
        row0_g = c * ROWS_CORE + row0_l
        chunk0 = row0_g * L // CHUNK

        @pl.loop(0, CHUNK)
        def _(i):
            @pl.loop(0, D, step=LANES)
            def _(j):
                rows_v[i, pl.ds(j, LANES)] = jnp.zeros((LANES,), jnp.float32)

        @pl.loop(0, ROWS_W, step=CHUNK)
        def _(r):
            pltpu.sync_copy(rows_v, acc_sh.at[pl.ds(row0_l + r, CHUNK)])

        plsc.subcore_barrier()

        @pl.loop(0, NBLK)
        def _(blk):
            blk_chunk0 = chunk0 + blk * IDXBLK
            pltpu.sync_copy(ids_hbm.at[pl.ds(blk_chunk0, IDXBLK)], tidx_v)

            @pl.loop(0, IDXBLK)
            def _(j):
                t0 = (blk_chunk0 + j) * CHUNK - c * (ROWS_CORE * L)

                @pl.loop(0, CHUNK, step=LANES)
                def _(i):
                    tv = t0 + i + lax.iota(jnp.int32, LANES)
                    didx_v[pl.ds(i, LANES)] = tv // L

                pltpu.sync_copy(table_hbm.at[tidx_v.at[j]], rows_v)
                pltpu.sync_copy(rows_v, acc_sh.at[didx_v], add=True)

        plsc.subcore_barrier()

        @pl.loop(0, ROWS_W, step=CHUNK)
        def _(r):
            pltpu.sync_copy(acc_sh.at[pl.ds(row0_l + r, CHUNK)], rows_v)

            @pl.loop(0, CHUNK)
            def _(i):
                @pl.loop(0, D, step=LANES)
                def _(j):
                    rows_v[i, pl.ds(j, LANES)] = (
                        rows_v[i, pl.ds(j, LANES)] * INV_L
                    )

            pltpu.sync_copy(rows_v, out_hbm.at[pl.ds(row0_g + r, CHUNK)])

    return k(ids2d, table)


def kernel(input_ids, table):
    ids2d = input_ids.astype(jnp.int32).reshape(B * L // CHUNK, CHUNK)
    return _sc_embed_mean(ids2d, table)


# speedup vs baseline: 10.4399x; 10.4399x over previous
import dataclasses
import functools

import jax
import jax.numpy as jnp
from jax import lax
from jax.experimental import pallas as pl
from jax.experimental.pallas import tpu as pltpu
from jax.experimental.pallas import tpu_sc as plsc

VOCAB = 30522
D = 64
B = 16384
L = 200

NC = 2
NS = 16
LANES = 16
NW = NC * NS
ROWS_W = B // NW
TOK_W = ROWS_W * L
CHUNK = 128
IDXBLK = 80
NBLK = TOK_W // (IDXBLK * CHUNK)
ROWS_CORE = B // NC
INV_L = 1.0 / L


def _compiler_params():
    cp = pltpu.CompilerParams()
    for field, val in (("needs_layout_passes", False),
                       ("use_tc_tiling_on_sc", False)):
        if field in pltpu.CompilerParams.__dataclass_fields__:
            cp = dataclasses.replace(cp, **{field: val})
    return cp


def _sc_embed_mean(ids2d, table):
    mesh = plsc.VectorSubcoreMesh(core_axis_name="c", subcore_axis_name="s")

    @functools.partial(
        pl.kernel,
        mesh=mesh,
        compiler_params=_compiler_params(),
        out_type=jax.ShapeDtypeStruct((B, D), jnp.float32),
        scratch_types=[
            pltpu.VMEM((IDXBLK, CHUNK), jnp.int32),
            pltpu.VMEM((CHUNK, D), jnp.float32),
            pltpu.VMEM((CHUNK,), jnp.int32),
            pltpu.VMEM_SHARED((ROWS_CORE, D), jnp.float32),
        ],
    )
    def k(ids_hbm, table_hbm, out_hbm, tidx_v, rows_v, didx_v, acc_sh):
        c = lax.axis_index("c")
        s = lax.axis_index("s")
        row0_l = s * ROWS_W
        row0_g = c * ROWS_CORE + row0_l
        chunk0 = row0_g * L // CHUNK

        @pl.loop(0, CHUNK)
        def _(i):
            @pl.loop(0, D, step=LANES)
            def _(j):
                rows_v[i, pl.ds(j, LANES)] = jnp.zeros((LANES,), jnp.float32)

        @pl.loop(0, ROWS_W, step=CHUNK)
        def _(r):
            pltpu.sync_copy(rows_v, acc_sh.at[pl.ds(row0_l + r, CHUNK)])

        plsc.subcore_barrier()

        @pl.loop(0, NBLK)
        def _(blk):
            blk_chunk0 = chunk0 + blk * IDXBLK
            pltpu.sync_copy(
                ids_hbm.at[pl.ds(pl.multiple_of(blk_chunk0, 8), IDXBLK)],
                tidx_v,
            )

            @pl.loop(0, IDXBLK)
            def _(j):
                t0 = (blk_chunk0 + j) * CHUNK - c * (ROWS_CORE * L)

                @pl.loop(0, CHUNK, step=LANES)
                def _(i):
                    tv = t0 + i + lax.iota(jnp.int32, LANES)
                    didx_v[pl.ds(i, LANES)] = tv // L

                pltpu.sync_copy(table_hbm.at[tidx_v.at[j]], rows_v)
                pltpu.sync_copy(rows_v, acc_sh.at[didx_v], add=True)

        plsc.subcore_barrier()

        @pl.loop(0, ROWS_W, step=CHUNK)
        def _(r):
            pltpu.sync_copy(acc_sh.at[pl.ds(row0_l + r, CHUNK)], rows_v)

            @pl.loop(0, CHUNK)
            def _(i):
                @pl.loop(0, D, step=LANES)
                def _(j):
                    rows_v[i, pl.ds(j, LANES)] = (
                        rows_v[i, pl.ds(j, LANES)] * INV_L
                    )

            pltpu.sync_copy(rows_v, out_hbm.at[pl.ds(row0_g + r, CHUNK)])

    return k(ids2d, table)


def kernel(input_ids, table):
    ids2d = input_ids.astype(jnp.int32).reshape(B * L // CHUNK, CHUNK)
    return _sc_embed_mean(ids2d, table)


# speedup vs baseline: 20.5544x; 1.9688x over previous
import dataclasses
import functools

import jax
import jax.numpy as jnp
from jax import lax
from jax.experimental import pallas as pl
from jax.experimental.pallas import tpu as pltpu
from jax.experimental.pallas import tpu_sc as plsc

VOCAB = 30522
D = 64
B = 16384
L = 200

NC = 2
NS = 16
LANES = 16
NW = NC * NS
ROWS_W = B // NW
TOK_W = ROWS_W * L
CHUNK = 128
IDXBLK = 80
NBLK = TOK_W // (IDXBLK * CHUNK)
ROWS_CORE = B // NC
INV_L = 1.0 / L
K = 4
RB = 2 * K
NG = IDXBLK // K


def _compiler_params():
    cp = pltpu.CompilerParams()
    for field, val in (("needs_layout_passes", False),
                       ("use_tc_tiling_on_sc", False)):
        if field in pltpu.CompilerParams.__dataclass_fields__:
            cp = dataclasses.replace(cp, **{field: val})
    return cp


def _sc_embed_mean(ids2d, table):
    mesh = plsc.VectorSubcoreMesh(core_axis_name="c", subcore_axis_name="s")

    @functools.partial(
        pl.kernel,
        mesh=mesh,
        compiler_params=_compiler_params(),
        out_type=jax.ShapeDtypeStruct((B, D), jnp.float32),
        scratch_types=[
            pltpu.VMEM((IDXBLK, CHUNK), jnp.int32),
            pltpu.VMEM((RB, CHUNK, D), jnp.float32),
            pltpu.VMEM((RB, CHUNK), jnp.int32),
            pltpu.VMEM_SHARED((ROWS_CORE, D), jnp.float32),
            pltpu.SemaphoreType.DMA,
            pltpu.SemaphoreType.DMA,
            pltpu.SemaphoreType.DMA,
            pltpu.SemaphoreType.DMA,
        ],
    )
    def k(ids_hbm, table_hbm, out_hbm, tidx_v, rows_v, didx_v, acc_sh,
          gsem0, gsem1, ssem0, ssem1):
        gsem = (gsem0, gsem1)
        ssem = (ssem0, ssem1)
        c = lax.axis_index("c")
        s = lax.axis_index("s")
        row0_l = s * ROWS_W
        row0_g = c * ROWS_CORE + row0_l
        chunk0 = row0_g * L // CHUNK

        @pl.loop(0, CHUNK)
        def _(i):
            @pl.loop(0, D, step=LANES)
            def _(j):
                rows_v[0, i, pl.ds(j, LANES)] = jnp.zeros((LANES,), jnp.float32)

        @pl.loop(0, ROWS_W, step=CHUNK)
        def _(r):
            pltpu.sync_copy(rows_v.at[0], acc_sh.at[pl.ds(row0_l + r, CHUNK)])

        plsc.subcore_barrier()

        def fill_didx(buf, chunk_idx):
            t0 = chunk_idx * CHUNK - c * (ROWS_CORE * L)

            @pl.loop(0, CHUNK, step=LANES)
            def _(i):
                tv = t0 + i + lax.iota(jnp.int32, LANES)
                didx_v[buf, pl.ds(i, LANES)] = tv // L

        def start_gather(buf, j, sem):
            pltpu.async_copy(table_hbm.at[tidx_v.at[j]], rows_v.at[buf], sem)

        def wait_gather(buf, j, sem):
            pltpu.make_async_copy(
                table_hbm.at[tidx_v.at[j]], rows_v.at[buf], sem
            ).wait()

        def start_scatter(buf, sem):
            pltpu.async_copy(
                rows_v.at[buf], acc_sh.at[didx_v.at[buf]], sem, add=True
            )

        def wait_scatter(buf, sem):
            pltpu.make_async_copy(
                rows_v.at[buf], acc_sh.at[didx_v.at[buf]], sem
            ).wait()

        @pl.loop(0, NBLK)
        def _(blk):
            blk_chunk0 = chunk0 + blk * IDXBLK
            pltpu.sync_copy(
                ids_hbm.at[pl.ds(pl.multiple_of(blk_chunk0, 8), IDXBLK)],
                tidx_v,
            )

            for half in (0, 1):
                for kk in range(K):
                    start_gather(half * K + kk, half * K + kk, gsem[half])

            @pl.loop(0, NG, step=2)
            def _(g0):
                for half in (0, 1):
                    g = g0 + half
                    base = half * K

                    for kk in range(K):
                        wait_gather(base + kk, g * K + kk, gsem[half])
                    for kk in range(K):
                        fill_didx(base + kk, blk_chunk0 + g * K + kk)
                        start_scatter(base + kk, ssem[half])
                    for kk in range(K):
                        wait_scatter(base + kk, ssem[half])

                    @pl.when(g0 < NG - 2)
                    def _():
                        for kk in range(K):
                            start_gather(
                                base + kk, (g + 2) * K + kk, gsem[half]
                            )

        plsc.subcore_barrier()

        @pl.loop(0, ROWS_W, step=CHUNK)
        def _(r):
            pltpu.sync_copy(acc_sh.at[pl.ds(row0_l + r, CHUNK)], rows_v.at[0])

            @pl.loop(0, CHUNK)
            def _(i):
                @pl.loop(0, D, step=LANES)
                def _(j):
                    rows_v[0, i, pl.ds(j, LANES)] = (
                        rows_v[0, i, pl.ds(j, LANES)] * INV_L
                    )

            pltpu.sync_copy(rows_v.at[0], out_hbm.at[pl.ds(row0_g + r, CHUNK)])

    return k(ids2d, table)


def kernel(input_ids, table):
    ids2d = input_ids.astype(jnp.int32).reshape(B * L // CHUNK, CHUNK)
    return _sc_embed_mean(ids2d, table)


# speedup vs baseline: 35.8358x; 1.7435x over previous
import dataclasses
import functools

import jax
import jax.numpy as jnp
from jax import lax
from jax.experimental import pallas as pl
from jax.experimental.pallas import tpu as pltpu
from jax.experimental.pallas import tpu_sc as plsc

VOCAB = 30522
D = 64
B = 16384
L = 200

NC = 2
NS = 16
LANES = 16
NW = NC * NS
ROWS_W = B // NW
R_STAGE = 128
NSTAGE = ROWS_W // R_STAGE
NB = 4
GL1 = 128
GL2 = L - GL1
INV_L = 1.0 / L


def _compiler_params():
    cp = pltpu.CompilerParams()
    for field, val in (("needs_layout_passes", False),
                       ("use_tc_tiling_on_sc", False)):
        if field in pltpu.CompilerParams.__dataclass_fields__:
            cp = dataclasses.replace(cp, **{field: val})
    return cp


def _sc_embed_mean(ids, table):
    mesh = plsc.VectorSubcoreMesh(core_axis_name="c", subcore_axis_name="s")

    @functools.partial(
        pl.kernel,
        mesh=mesh,
        compiler_params=_compiler_params(),
        out_type=jax.ShapeDtypeStruct((B, D), jnp.float32),
        scratch_types=[
            pltpu.VMEM((2, R_STAGE, L), jnp.int32),
            pltpu.VMEM((NB, L, D), jnp.float32),
            pltpu.VMEM((R_STAGE, D), jnp.float32),
            pltpu.SemaphoreType.DMA,
            pltpu.SemaphoreType.DMA,
            pltpu.SemaphoreType.DMA,
            pltpu.SemaphoreType.DMA,
        ],
    )
    def k(ids_hbm, table_hbm, out_hbm, ids_v, rbuf, obuf,
          gsem0, gsem1, gsem2, gsem3):
        gsem = (gsem0, gsem1, gsem2, gsem3)
        c = lax.axis_index("c")
        s = lax.axis_index("s")
        row0_g = (c * NS + s) * ROWS_W

        def load_stage(st):
            pltpu.sync_copy(
                ids_hbm.at[pl.ds(row0_g + st * R_STAGE, R_STAGE)],
                ids_v.at[st % 2],
            )

        def start_gathers(b, idh, rl):
            idrow = ids_v.at[idh]
            pltpu.async_copy(
                table_hbm.at[idrow.at[rl, pl.ds(0, GL1)]],
                rbuf.at[b, pl.ds(0, GL1)], gsem[b],
            )
            pltpu.async_copy(
                table_hbm.at[idrow.at[rl, pl.ds(GL1, GL2)]],
                rbuf.at[b, pl.ds(GL1, GL2)], gsem[b],
            )

        def wait_gathers(b, idh, rl):
            idrow = ids_v.at[idh]
            pltpu.make_async_copy(
                table_hbm.at[idrow.at[rl, pl.ds(0, GL1)]],
                rbuf.at[b, pl.ds(0, GL1)], gsem[b],
            ).wait()
            pltpu.make_async_copy(
                table_hbm.at[idrow.at[rl, pl.ds(GL1, GL2)]],
                rbuf.at[b, pl.ds(GL1, GL2)], gsem[b],
            ).wait()

        zero = jnp.zeros((LANES,), jnp.float32)

        def process(st, rl, b, lookahead):
            wait_gathers(b, st % 2, rl)

            def acc_body(t, acc):
                return tuple(
                    acc[q] + rbuf[b, t, pl.ds(q * LANES, LANES)]
                    for q in range(4)
                )

            accs = plsc.parallel_loop(
                0, L, unroll=4, carry=(zero, zero, zero, zero)
            )(acc_body)

            for q in range(4):
                obuf[rl, pl.ds(q * LANES, LANES)] = accs[q] * INV_L

            if lookahead is not None:
                nidh, nrl = lookahead
                start_gathers(b, nidh, nrl)

        load_stage(0)
        if NSTAGE > 1:
            load_stage(1)
        for b in range(NB):
            start_gathers(b, 0, b)

        for st in range(NSTAGE):
            if 1 <= st and st + 1 < NSTAGE:
                load_stage(st + 1)

            @pl.loop(0, R_STAGE - NB, step=NB)
            def _(rl0):
                for bb in range(NB):
                    process(st, rl0 + bb, bb, (st % 2, rl0 + bb + NB))

            for bb in range(NB):
                rl = R_STAGE - NB + bb
                la = ((st + 1) % 2, bb) if st + 1 < NSTAGE else None
                process(st, rl, bb, la)

            pltpu.sync_copy(
                obuf, out_hbm.at[pl.ds(row0_g + st * R_STAGE, R_STAGE)]
            )

    return k(ids, table)


def kernel(input_ids, table):
    return _sc_embed_mean(input_ids.astype(jnp.int32), table)


# speedup vs baseline: 46.7882x; 1.3056x over previous
import dataclasses
import functools

import jax
import jax.numpy as jnp
from jax import lax
from jax.experimental import pallas as pl
from jax.experimental.pallas import tpu as pltpu
from jax.experimental.pallas import tpu_sc as plsc

VOCAB = 30522
D = 64
B = 16384
L = 200

NC = 2
NS = 16
LANES = 16
NW = NC * NS
ROWS_W = B // NW
R_STAGE = 128
NSTAGE = ROWS_W // R_STAGE
NB = 4
GL1 = 128
GL2 = L - GL1
INV_L = 1.0 / L


def _compiler_params():
    cp = pltpu.CompilerParams()
    for field, val in (("needs_layout_passes", False),
                       ("use_tc_tiling_on_sc", False)):
        if field in pltpu.CompilerParams.__dataclass_fields__:
            cp = dataclasses.replace(cp, **{field: val})
    return cp


def _sc_embed_mean(ids, table):
    mesh = plsc.VectorSubcoreMesh(core_axis_name="c", subcore_axis_name="s")

    @functools.partial(
        pl.kernel,
        mesh=mesh,
        compiler_params=_compiler_params(),
        out_type=jax.ShapeDtypeStruct((B, D), jnp.float32),
        scratch_types=[
            pltpu.VMEM((2, R_STAGE, L), jnp.int32),
            pltpu.VMEM((NB, L, D), jnp.bfloat16),
            pltpu.VMEM((R_STAGE, D), jnp.float32),
            pltpu.SemaphoreType.DMA,
            pltpu.SemaphoreType.DMA,
            pltpu.SemaphoreType.DMA,
            pltpu.SemaphoreType.DMA,
        ],
    )
    def k(ids_hbm, table_hbm, out_hbm, ids_v, rbuf, obuf,
          gsem0, gsem1, gsem2, gsem3):
        gsem = (gsem0, gsem1, gsem2, gsem3)
        c = lax.axis_index("c")
        s = lax.axis_index("s")
        row0_g = (c * NS + s) * ROWS_W

        def load_stage(st):
            pltpu.sync_copy(
                ids_hbm.at[pl.ds(row0_g + st * R_STAGE, R_STAGE)],
                ids_v.at[st % 2],
            )

        def start_gathers(b, idh, rl):
            idrow = ids_v.at[idh]
            pltpu.async_copy(
                table_hbm.at[idrow.at[rl, pl.ds(0, GL1)]],
                rbuf.at[b, pl.ds(0, GL1)], gsem[b],
            )
            pltpu.async_copy(
                table_hbm.at[idrow.at[rl, pl.ds(GL1, GL2)]],
                rbuf.at[b, pl.ds(GL1, GL2)], gsem[b],
            )

        def wait_gathers(b, idh, rl):
            idrow = ids_v.at[idh]
            pltpu.make_async_copy(
                table_hbm.at[idrow.at[rl, pl.ds(0, GL1)]],
                rbuf.at[b, pl.ds(0, GL1)], gsem[b],
            ).wait()
            pltpu.make_async_copy(
                table_hbm.at[idrow.at[rl, pl.ds(GL1, GL2)]],
                rbuf.at[b, pl.ds(GL1, GL2)], gsem[b],
            ).wait()

        zero = jnp.zeros((LANES,), jnp.float32)

        def process(st, rl, b, lookahead):
            wait_gathers(b, st % 2, rl)

            def acc_body(t, acc):
                x0 = rbuf[b, t, pl.ds(0, 2 * LANES)]
                x1 = rbuf[b, t, pl.ds(2 * LANES, 2 * LANES)]
                a0, a1 = plsc.unpack(x0, format=plsc.PackFormat.INTERLEAVED)
                a2, a3 = plsc.unpack(x1, format=plsc.PackFormat.INTERLEAVED)
                return (acc[0] + a0, acc[1] + a1, acc[2] + a2, acc[3] + a3)

            accs = plsc.parallel_loop(
                0, L, unroll=4, carry=(zero, zero, zero, zero)
            )(acc_body)

            for q in range(4):
                obuf[rl, pl.ds(q * LANES, LANES)] = accs[q] * INV_L

            if lookahead is not None:
                nidh, nrl = lookahead
                start_gathers(b, nidh, nrl)

        load_stage(0)
        if NSTAGE > 1:
            load_stage(1)
        for b in range(NB):
            start_gathers(b, 0, b)

        for st in range(NSTAGE):
            if 1 <= st and st + 1 < NSTAGE:
                load_stage(st + 1)

            @pl.loop(0, R_STAGE - NB, step=NB)
            def _(rl0):
                for bb in range(NB):
                    process(st, rl0 + bb, bb, (st % 2, rl0 + bb + NB))

            for bb in range(NB):
                rl = R_STAGE - NB + bb
                la = ((st + 1) % 2, bb) if st + 1 < NSTAGE else None
                process(st, rl, bb, la)

            pltpu.sync_copy(
                obuf, out_hbm.at[pl.ds(row0_g + st * R_STAGE, R_STAGE)]
            )

    return k(ids, table)


def kernel(input_ids, table):
    table_pre = (
        table.reshape(VOCAB, 2, 2, LANES)
        .transpose(0, 1, 3, 2)
        .reshape(VOCAB, D)
        .astype(jnp.bfloat16)
    )
    return _sc_embed_mean(input_ids.astype(jnp.int32), table_pre)


# speedup vs baseline: 51.0451x; 1.0910x over previous
import dataclasses
import functools

import jax
import jax.numpy as jnp
from jax import lax
from jax.experimental import pallas as pl
from jax.experimental.pallas import tpu as pltpu
from jax.experimental.pallas import tpu_sc as plsc

VOCAB = 30522
D = 64
B = 16384
L = 200

NC = 2
NS = 16
LANES = 16
NW = NC * NS
ROWS_W = B // NW
R_STAGE = 128
NSTAGE = ROWS_W // R_STAGE
NB = 8
GL1 = 128
GL2 = L - GL1
INV_L = 1.0 / L


def _compiler_params():
    cp = pltpu.CompilerParams()
    for field, val in (("needs_layout_passes", False),
                       ("use_tc_tiling_on_sc", False)):
        if field in pltpu.CompilerParams.__dataclass_fields__:
            cp = dataclasses.replace(cp, **{field: val})
    return cp


def _sc_embed_mean(ids, table):
    mesh = plsc.VectorSubcoreMesh(core_axis_name="c", subcore_axis_name="s")

    @functools.partial(
        pl.kernel,
        mesh=mesh,
        compiler_params=_compiler_params(),
        out_type=jax.ShapeDtypeStruct((B, D), jnp.float32),
        scratch_types=[
            pltpu.VMEM((2, R_STAGE, L), jnp.int32),
            pltpu.VMEM((NB, L, D), jnp.bfloat16),
            pltpu.VMEM((R_STAGE, D), jnp.float32),
        ] + [pltpu.SemaphoreType.DMA] * NB,
    )
    def k(ids_hbm, table_hbm, out_hbm, ids_v, rbuf, obuf, *gsem):
        c = lax.axis_index("c")
        s = lax.axis_index("s")
        row0_g = (c * NS + s) * ROWS_W

        def load_stage(st):
            pltpu.sync_copy(
                ids_hbm.at[pl.ds(row0_g + st * R_STAGE, R_STAGE)],
                ids_v.at[st % 2],
            )

        def start_gathers(b, idh, rl):
            idrow = ids_v.at[idh]
            pltpu.async_copy(
                table_hbm.at[idrow.at[rl, pl.ds(0, GL1)]],
                rbuf.at[b, pl.ds(0, GL1)], gsem[b],
            )
            pltpu.async_copy(
                table_hbm.at[idrow.at[rl, pl.ds(GL1, GL2)]],
                rbuf.at[b, pl.ds(GL1, GL2)], gsem[b],
            )

        def wait_gathers(b, idh, rl):
            idrow = ids_v.at[idh]
            pltpu.make_async_copy(
                table_hbm.at[idrow.at[rl, pl.ds(0, GL1)]],
                rbuf.at[b, pl.ds(0, GL1)], gsem[b],
            ).wait()
            pltpu.make_async_copy(
                table_hbm.at[idrow.at[rl, pl.ds(GL1, GL2)]],
                rbuf.at[b, pl.ds(GL1, GL2)], gsem[b],
            ).wait()

        zero = jnp.zeros((LANES,), jnp.float32)

        def process(st, rl, b, lookahead):
            wait_gathers(b, st % 2, rl)

            def acc_body(t, acc):
                x0 = rbuf[b, t, pl.ds(0, 2 * LANES)]
                x1 = rbuf[b, t, pl.ds(2 * LANES, 2 * LANES)]
                y0 = rbuf[b, t + 1, pl.ds(0, 2 * LANES)]
                y1 = rbuf[b, t + 1, pl.ds(2 * LANES, 2 * LANES)]
                s0 = x0 + y0
                s1 = x1 + y1
                a0, a1 = plsc.unpack(s0, format=plsc.PackFormat.INTERLEAVED)
                a2, a3 = plsc.unpack(s1, format=plsc.PackFormat.INTERLEAVED)
                return (acc[0] + a0, acc[1] + a1, acc[2] + a2, acc[3] + a3)

            accs = plsc.parallel_loop(
                0, L, step=2, unroll=4, carry=(zero, zero, zero, zero)
            )(acc_body)

            for q in range(4):
                obuf[rl, pl.ds(q * LANES, LANES)] = accs[q] * INV_L

            if lookahead is not None:
                nidh, nrl = lookahead
                start_gathers(b, nidh, nrl)

        load_stage(0)
        if NSTAGE > 1:
            load_stage(1)
        for b in range(NB):
            start_gathers(b, 0, b)

        for st in range(NSTAGE):
            if 1 <= st and st + 1 < NSTAGE:
                load_stage(st + 1)

            @pl.loop(0, R_STAGE - NB, step=NB)
            def _(rl0):
                for bb in range(NB):
                    process(st, rl0 + bb, bb, (st % 2, rl0 + bb + NB))

            for bb in range(NB):
                rl = R_STAGE - NB + bb
                la = ((st + 1) % 2, bb) if st + 1 < NSTAGE else None
                process(st, rl, bb, la)

            pltpu.sync_copy(
                obuf, out_hbm.at[pl.ds(row0_g + st * R_STAGE, R_STAGE)]
            )

    return k(ids, table)


def kernel(input_ids, table):
    table_pre = (
        table.reshape(VOCAB, 2, 2, LANES)
        .transpose(0, 1, 3, 2)
        .reshape(VOCAB, D)
        .astype(jnp.bfloat16)
    )
    return _sc_embed_mean(input_ids.astype(jnp.int32), table_pre)
